# baseline (speedup 1.0000x reference)
import functools

import jax
import jax.numpy as jnp
from jax import lax
from jax.experimental import pallas as pl
from jax.experimental.pallas import tpu as pltpu
from jax.experimental.pallas import tpu_sc as plsc

NUM_LEVELS = 8
TOKEN_DIM = 1024
TOTAL_TOK = 32768

NC, NS, L = 2, 16, 16
NW = NC * NS
TOK_PER_W = TOTAL_TOK // NW
CHUNK = 8
NCHUNK = TOK_PER_W // CHUNK
GROWS = NUM_LEVELS * CHUNK
PACKED_DIM = TOKEN_DIM // 2
WVECS = PACKED_DIM // L


def _sc_embed(xi_prep, cat_tab):
    mesh = plsc.VectorSubcoreMesh(core_axis_name="c", subcore_axis_name="s")

    @functools.partial(
        pl.kernel,
        out_type=jax.ShapeDtypeStruct((TOTAL_TOK, TOKEN_DIM), jnp.float32),
        mesh=mesh,
        scratch_types=[
            pltpu.VMEM((NCHUNK, GROWS), jnp.int32),
            pltpu.VMEM((GROWS, PACKED_DIM), jnp.int32),
            pltpu.VMEM((GROWS, PACKED_DIM), jnp.int32),
            pltpu.VMEM((CHUNK, TOKEN_DIM), jnp.float32),
            pltpu.VMEM((CHUNK, TOKEN_DIM), jnp.float32),
            pltpu.SemaphoreType.DMA,
            pltpu.SemaphoreType.DMA,
            pltpu.SemaphoreType.DMA,
            pltpu.SemaphoreType.DMA,
        ],
    )
    def k(xi_hbm, tab_hbm, out_hbm,
          idx_v, sb0, sb1, ost0, ost1, gsem0, gsem1, osem0, osem1):
        sbs = (sb0, sb1)
        osts = (ost0, ost1)
        gsems = (gsem0, gsem1)
        osems = (osem0, osem1)

        c = lax.axis_index("c")
        s = lax.axis_index("s")
        wid = s * NC + c
        tok0 = wid * TOK_PER_W
        chk0 = wid * NCHUNK

        pltpu.sync_copy(xi_hbm.at[pl.ds(chk0, NCHUNK)], idx_v)

        def gather_desc(j_dyn, p):
            return pltpu.make_async_copy(
                tab_hbm.at[idx_v.at[j_dyn]], sbs[p], gsems[p])

        hi_mask = jnp.int32(-65536)

        def lane_pass(p):
            sb, ost = sbs[p], osts[p]

            @plsc.parallel_loop(0, CHUNK * WVECS, unroll=8)
            def _(i):
                t = i >> 5
                col = (i & (WVECS - 1)) * L
                w = sb[t, pl.ds(col, L)]
                acc_lo = lax.bitcast_convert_type(w << 16, jnp.float32)
                acc_hi = lax.bitcast_convert_type(w & hi_mask, jnp.float32)
                for lv in range(1, NUM_LEVELS):
                    w = sb[lv * CHUNK + t, pl.ds(col, L)]
                    acc_lo = acc_lo + lax.bitcast_convert_type(
                        w << 16, jnp.float32)
                    acc_hi = acc_hi + lax.bitcast_convert_type(
                        w & hi_mask, jnp.float32)
                ost[t, pl.ds(col * 2, L)] = acc_lo
                ost[t, pl.ds(col * 2 + L, L)] = acc_hi

        def out_desc(j_dyn, p):
            return pltpu.make_async_copy(
                osts[p], out_hbm.at[pl.ds(tok0 + j_dyn * CHUNK, CHUNK)],
                osems[p])

        def chunk(j_dyn, p, drain, prefetch):
            gather_desc(j_dyn, p).wait()
            if drain:
                out_desc(j_dyn, p).wait()
            lane_pass(p)
            out_desc(j_dyn, p).start()
            if prefetch:
                gather_desc(j_dyn + 2, p).start()

        gather_desc(0, 0).start()
        gather_desc(1, 1).start()
        chunk(0, 0, drain=False, prefetch=True)
        chunk(1, 1, drain=False, prefetch=True)

        def body(t_it, carry):
            chunk(2 * t_it, 0, drain=True, prefetch=True)
            chunk(2 * t_it + 1, 1, drain=True, prefetch=True)
            return carry

        lax.fori_loop(1, NCHUNK // 2 - 1, body, 0)

        chunk(NCHUNK - 2, 0, drain=True, prefetch=False)
        chunk(NCHUNK - 1, 1, drain=True, prefetch=False)

        out_desc(NCHUNK - 2, 0).wait()
        out_desc(NCHUNK - 1, 1).wait()

    return k(xi_prep, cat_tab)


def _pack_table(t):
    t16 = t.astype(jnp.bfloat16)
    sh = t16.reshape(t.shape[0], t.shape[1] // 32, 2, 16)
    sh = sh.transpose(0, 1, 3, 2)
    return lax.bitcast_convert_type(sh, jnp.int32).reshape(
        t.shape[0], t.shape[1] // 2)


def kernel(xi, table0, table1, table2, table3, table4, table5, table6,
           table7):
    tabs = (table0, table1, table2, table3, table4, table5, table6, table7)
    cat_tab = jnp.concatenate([_pack_table(t) for t in tabs], axis=0)
    xib = xi + jnp.arange(NUM_LEVELS, dtype=xi.dtype)[None, :] * 1024
    xi_prep = xib.reshape(TOTAL_TOK // CHUNK, CHUNK, NUM_LEVELS)
    xi_prep = xi_prep.transpose(0, 2, 1).reshape(TOTAL_TOK // CHUNK, GROWS)
    return _sc_embed(xi_prep, cat_tab)


# speedup vs baseline: 1.1871x; 1.1871x over previous
import functools

import jax
import jax.numpy as jnp
from jax import lax
from jax.experimental import pallas as pl
from jax.experimental.pallas import tpu as pltpu
from jax.experimental.pallas import tpu_sc as plsc

NUM_LEVELS = 8
TOKEN_DIM = 1024
TOTAL_TOK = 32768

NC, NS, L = 2, 16, 16
NW = NC * NS
TOK_PER_W = TOTAL_TOK // NW
CHUNK = 8
NCHUNK = TOK_PER_W // CHUNK
PACKED_DIM = TOKEN_DIM // 2
WVECS = PACKED_DIM // L
NSB = 3


def _sc_embed(xiT, *ptabs_args):
    mesh = plsc.VectorSubcoreMesh(core_axis_name="c", subcore_axis_name="s")

    @functools.partial(
        pl.kernel,
        out_type=jax.ShapeDtypeStruct((TOTAL_TOK, TOKEN_DIM), jnp.float32),
        mesh=mesh,
        scratch_types=[
            pltpu.VMEM((NUM_LEVELS, TOK_PER_W), jnp.int32),
            pltpu.VMEM((NUM_LEVELS, CHUNK, PACKED_DIM), jnp.int32),
            pltpu.VMEM((NUM_LEVELS, CHUNK, PACKED_DIM), jnp.int32),
            pltpu.VMEM((NUM_LEVELS, CHUNK, PACKED_DIM), jnp.int32),
            pltpu.VMEM((CHUNK, TOKEN_DIM), jnp.float32),
            pltpu.VMEM((CHUNK, TOKEN_DIM), jnp.float32),
            pltpu.SemaphoreType.DMA,
            pltpu.SemaphoreType.DMA,
            pltpu.SemaphoreType.DMA,
            pltpu.SemaphoreType.DMA,
            pltpu.SemaphoreType.DMA,
        ],
    )
    def k(xiT_hbm, p0, p1, p2, p3, p4, p5, p6, p7, out_hbm,
          idx_v, sb0, sb1, sb2, ost0, ost1,
          gsem0, gsem1, gsem2, osem0, osem1):
        ptabs = (p0, p1, p2, p3, p4, p5, p6, p7)
        sbs = (sb0, sb1, sb2)
        osts = (ost0, ost1)
        gsems = (gsem0, gsem1, gsem2)
        osems = (osem0, osem1)

        c = lax.axis_index("c")
        s = lax.axis_index("s")
        wid = s * NC + c
        tok0 = wid * TOK_PER_W

        pltpu.sync_copy(xiT_hbm.at[:, pl.ds(tok0, TOK_PER_W)], idx_v)

        def gather_desc(lv, j_dyn, p):
            return pltpu.make_async_copy(
                ptabs[lv].at[idx_v.at[lv, pl.ds(j_dyn * CHUNK, CHUNK)]],
                sbs[p].at[lv], gsems[p])

        def issue_gathers(j_dyn, p):
            for lv in range(NUM_LEVELS):
                gather_desc(lv, j_dyn, p).start()

        def wait_gathers(j_dyn, p):
            for lv in range(NUM_LEVELS):
                gather_desc(lv, j_dyn, p).wait()

        hi_mask = jnp.int32(-65536)

        def lane_pass(p, q):
            sb, ost = sbs[p], osts[q]

            @plsc.parallel_loop(0, CHUNK * WVECS, unroll=8)
            def _(i):
                t = i >> 5
                col = (i & (WVECS - 1)) * L
                w = sb[0, t, pl.ds(col, L)]
                acc_lo = lax.bitcast_convert_type(w << 16, jnp.float32)
                acc_hi = lax.bitcast_convert_type(w & hi_mask, jnp.float32)
                for lv in range(1, NUM_LEVELS):
                    w = sb[lv, t, pl.ds(col, L)]
                    acc_lo = acc_lo + lax.bitcast_convert_type(
                        w << 16, jnp.float32)
                    acc_hi = acc_hi + lax.bitcast_convert_type(
                        w & hi_mask, jnp.float32)
                ost[t, pl.ds(col * 2, L)] = acc_lo
                ost[t, pl.ds(col * 2 + L, L)] = acc_hi

        def out_desc(j_dyn, q):
            return pltpu.make_async_copy(
                osts[q], out_hbm.at[pl.ds(tok0 + j_dyn * CHUNK, CHUNK)],
                osems[q])

        def chunk(j_dyn, p, q, drain, prefetch):
            wait_gathers(j_dyn, p)
            if drain:
                out_desc(j_dyn, q).wait()
            lane_pass(p, q)
            out_desc(j_dyn, q).start()
            if prefetch:
                issue_gathers(j_dyn + NSB, p)

        issue_gathers(0, 0)
        issue_gathers(1, 1)
        issue_gathers(2, 2)
        for j in range(6):
            chunk(j, j % NSB, j % 2, drain=(j >= 2), prefetch=True)

        def body(t_it, carry):
            j0 = 6 * t_it
            for r in range(6):
                chunk(j0 + r, r % NSB, r % 2, drain=True, prefetch=True)
            return carry

        lax.fori_loop(1, 120 // 6, body, 0)

        for j in range(120, NCHUNK):
            chunk(j, j % NSB, j % 2, drain=True, prefetch=(j + NSB < NCHUNK))

        out_desc(NCHUNK - 2, 0).wait()
        out_desc(NCHUNK - 1, 1).wait()

    return k(xiT, *ptabs_args)


def _pack_table(t):
    t16 = t.astype(jnp.bfloat16)
    sh = t16.reshape(t.shape[0], t.shape[1] // 32, 2, 16)
    sh = sh.transpose(0, 1, 3, 2)
    return lax.bitcast_convert_type(sh, jnp.int32).reshape(
        t.shape[0], t.shape[1] // 2)


def kernel(xi, table0, table1, table2, table3, table4, table5, table6,
           table7):
    xiT = xi.T
    packed = [_pack_table(t) for t in (table0, table1, table2, table3,
                                       table4, table5, table6, table7)]
    return _sc_embed(xiT, *packed)


# speedup vs baseline: 1.2751x; 1.0742x over previous
import functools

import jax
import jax.numpy as jnp
from jax import lax
from jax.experimental import pallas as pl
from jax.experimental.pallas import tpu as pltpu
from jax.experimental.pallas import tpu_sc as plsc

NUM_LEVELS = 8
TOKEN_DIM = 1024
TOTAL_TOK = 32768

NC, NS, L = 2, 16, 16
NW = NC * NS
TOK_PER_W = TOTAL_TOK // NW
CHUNK = 8
NCHUNK = TOK_PER_W // CHUNK
PACKED_DIM = TOKEN_DIM // 2
WVECS = PACKED_DIM // L
NSB = 3


def _sc_embed(xiT, *ptabs_args):
    mesh = plsc.VectorSubcoreMesh(core_axis_name="c", subcore_axis_name="s")

    @functools.partial(
        pl.kernel,
        out_type=jax.ShapeDtypeStruct((TOTAL_TOK, TOKEN_DIM), jnp.float32),
        mesh=mesh,
        scratch_types=[
            pltpu.VMEM((NUM_LEVELS, TOK_PER_W), jnp.int32),
            pltpu.VMEM((NUM_LEVELS, CHUNK, PACKED_DIM), jnp.int32),
            pltpu.VMEM((NUM_LEVELS, CHUNK, PACKED_DIM), jnp.int32),
            pltpu.VMEM((NUM_LEVELS, CHUNK, PACKED_DIM), jnp.int32),
            pltpu.VMEM((CHUNK, TOKEN_DIM), jnp.float32),
            pltpu.VMEM((CHUNK, TOKEN_DIM), jnp.float32),
            pltpu.SemaphoreType.DMA,
            pltpu.SemaphoreType.DMA,
            pltpu.SemaphoreType.DMA,
            pltpu.SemaphoreType.DMA,
            pltpu.SemaphoreType.DMA,
        ],
    )
    def k(xiT_hbm, p0, p1, p2, p3, p4, p5, p6, p7, out_hbm,
          idx_v, sb0, sb1, sb2, ost0, ost1,
          gsem0, gsem1, gsem2, osem0, osem1):
        ptabs = (p0, p1, p2, p3, p4, p5, p6, p7)
        sbs = (sb0, sb1, sb2)
        osts = (ost0, ost1)
        gsems = (gsem0, gsem1, gsem2)
        osems = (osem0, osem1)

        c = lax.axis_index("c")
        s = lax.axis_index("s")
        wid = s * NC + c
        tok0 = wid * TOK_PER_W

        pltpu.sync_copy(xiT_hbm.at[:, pl.ds(tok0, TOK_PER_W)], idx_v)

        def gather_desc(lv, j_dyn, p):
            return pltpu.make_async_copy(
                ptabs[lv].at[idx_v.at[lv, pl.ds(j_dyn * CHUNK, CHUNK)]],
                sbs[p].at[lv], gsems[p])

        def issue_gathers(j_dyn, p):
            for lv in range(NUM_LEVELS):
                gather_desc(lv, j_dyn, p).start()

        def wait_gathers(j_dyn, p):
            for lv in range(NUM_LEVELS):
                gather_desc(lv, j_dyn, p).wait()

        hi_mask = jnp.int32(-65536)

        def lane_pass(p, q):
            sb, ost = sbs[p], osts[q]

            @plsc.parallel_loop(0, CHUNK * WVECS, unroll=8)
            def _(i):
                t = i >> 5
                col = (i & (WVECS - 1)) * L
                w = sb[0, t, pl.ds(col, L)]
                acc_lo = lax.bitcast_convert_type(w << 16, jnp.float32)
                acc_hi = lax.bitcast_convert_type(w & hi_mask, jnp.float32)
                for lv in range(1, NUM_LEVELS):
                    w = sb[lv, t, pl.ds(col, L)]
                    acc_lo = acc_lo + lax.bitcast_convert_type(
                        w << 16, jnp.float32)
                    acc_hi = acc_hi + lax.bitcast_convert_type(
                        w & hi_mask, jnp.float32)
                ost[t, pl.ds(col, L)] = acc_lo
                ost[t, pl.ds(col + TOKEN_DIM // 2, L)] = acc_hi

        def out_desc(j_dyn, q):
            return pltpu.make_async_copy(
                osts[q], out_hbm.at[pl.ds(tok0 + j_dyn * CHUNK, CHUNK)],
                osems[q])

        def chunk(j_dyn, p, q, drain, prefetch):
            wait_gathers(j_dyn, p)
            if drain:
                out_desc(j_dyn, q).wait()
            lane_pass(p, q)
            out_desc(j_dyn, q).start()
            if prefetch:
                issue_gathers(j_dyn + NSB, p)

        issue_gathers(0, 0)
        issue_gathers(1, 1)
        issue_gathers(2, 2)
        for j in range(6):
            chunk(j, j % NSB, j % 2, drain=(j >= 2), prefetch=True)

        def body(t_it, carry):
            j0 = 6 * t_it
            for r in range(6):
                chunk(j0 + r, r % NSB, r % 2, drain=True, prefetch=True)
            return carry

        lax.fori_loop(1, 120 // 6, body, 0)

        for j in range(120, NCHUNK):
            chunk(j, j % NSB, j % 2, drain=True, prefetch=(j + NSB < NCHUNK))

        out_desc(NCHUNK - 2, 0).wait()
        out_desc(NCHUNK - 1, 1).wait()

    return k(xiT, *ptabs_args)


def _pack_table(t):
    h = t.shape[1] // 2
    u = lax.bitcast_convert_type(t.astype(jnp.bfloat16), jnp.uint16)
    u = u.astype(jnp.uint32)
    return lax.bitcast_convert_type(u[:, :h] | (u[:, h:] << 16), jnp.int32)


def kernel(xi, table0, table1, table2, table3, table4, table5, table6,
           table7):
    xiT = xi.T
    packed = [_pack_table(t) for t in (table0, table1, table2, table3,
                                       table4, table5, table6, table7)]
    return _sc_embed(xiT, *packed)
